# baseline (speedup 1.0000x reference)
import functools

import jax
import jax.numpy as jnp
from jax import lax
from jax.experimental import pallas as pl
from jax.experimental.pallas import tpu as pltpu
from jax.experimental.pallas import tpu_sc as plsc

N = 10000
E = 160000
D = 128
DE = 16
G = 256
NL = 4
LQ = 1024

NPAD = 10240
EPAD = 163840
NCORE = 2
NSUB = 16
NW = NCORE * NSUB
ROWS_PER_SUB = NPAD // NSUB
EDGES_PER_W = EPAD // NW
ECHUNK = 80
NCHUNK = EDGES_PER_W // ECHUNK
GPW = G // NW
BROWS = NPAD // NW

_MESH = dict(core_axis_name="c", subcore_axis_name="s")
_SC_PARAMS = pltpu.CompilerParams(needs_layout_passes=False)


def _enc_body(x_ref, we_ref, be_ref, wi_ref, bi_ref, xf_ref, y0_ref):
    xf = jnp.maximum(
        jnp.dot(x_ref[...], we_ref[...], preferred_element_type=jnp.float32)
        + be_ref[...], 0.0)
    xf_ref[...] = xf
    y0_ref[...] = (
        jnp.dot(xf, wi_ref[...], preferred_element_type=jnp.float32)
        + bi_ref[...])


def _encoder(x, We, be, Wi, bi):
    blk = 512
    return pl.pallas_call(
        _enc_body,
        grid=(NPAD // blk,),
        in_specs=[
            pl.BlockSpec((blk, D), lambda i: (i, 0)),
            pl.BlockSpec((D, D), lambda i: (0, 0)),
            pl.BlockSpec((1, D), lambda i: (0, 0)),
            pl.BlockSpec((D, D), lambda i: (0, 0)),
            pl.BlockSpec((1, D), lambda i: (0, 0)),
        ],
        out_specs=[pl.BlockSpec((blk, D), lambda i: (i, 0)),
                   pl.BlockSpec((blk, D), lambda i: (i, 0))],
        out_shape=[jax.ShapeDtypeStruct((NPAD, D), jnp.float32),
                   jax.ShapeDtypeStruct((NPAD, D), jnp.float32)],
    )(x, We, be, Wi, bi)


def _edge_mm_body(ea_ref, w_ref, b_ref, e_ref):
    e_ref[...] = (
        jnp.dot(ea_ref[...], w_ref[...], preferred_element_type=jnp.float32)
        + b_ref[...])


def _edge_mm(ea, W, b):
    blk = 2048
    return pl.pallas_call(
        _edge_mm_body,
        grid=(EPAD // blk,),
        in_specs=[
            pl.BlockSpec((blk, DE), lambda i: (i, 0)),
            pl.BlockSpec((DE, D), lambda i: (0, 0)),
            pl.BlockSpec((1, D), lambda i: (0, 0)),
        ],
        out_specs=pl.BlockSpec((blk, D), lambda i: (i, 0)),
        out_shape=jax.ShapeDtypeStruct((EPAD, D), jnp.float32),
    )(ea, W, b)


def _node_body(s_ref, xf_ref, p_ref, w1_ref, b1_ref, w2_ref, b2_ref,
               o_ref):
    z = s_ref[0] * xf_ref[...] + p_ref[...]
    h = jnp.maximum(
        jnp.dot(z, w1_ref[...], preferred_element_type=jnp.float32)
        + b1_ref[...], 0.0)
    o_ref[...] = jnp.maximum(
        jnp.dot(h, w2_ref[...], preferred_element_type=jnp.float32)
        + b2_ref[...], 0.0)


def _node_mlp(s, xf, aggr, W1f, b1f, W2f, b2f):
    blk = 512
    nblk = NPAD // blk
    return pl.pallas_call(
        _node_body,
        grid=(nblk,),
        in_specs=[
            pl.BlockSpec(memory_space=pltpu.SMEM),
            pl.BlockSpec((blk, D), lambda i: (i, 0)),
            pl.BlockSpec((blk, D), lambda i: (i, 0)),
            pl.BlockSpec((D, D), lambda i: (0, 0)),
            pl.BlockSpec((1, D), lambda i: (0, 0)),
            pl.BlockSpec((D, D), lambda i: (0, 0)),
            pl.BlockSpec((1, D), lambda i: (0, 0)),
        ],
        out_specs=pl.BlockSpec((blk, D), lambda i: (i, 0)),
        out_shape=jax.ShapeDtypeStruct((NPAD, D), jnp.float32),
    )(s, xf, aggr, W1f, b1f, W2f, b2f)


def _head_body(p0_ref, p1_ref, p2_ref, p3_ref, p4_ref, wl_ref, blin_ref,
               wp1_ref, bp1_ref, wp2_ref, bp2_ref, o_ref):
    out = p0_ref[...]
    for l, pref in enumerate((p1_ref, p2_ref, p3_ref, p4_ref)):
        out = out + (jnp.dot(pref[...], wl_ref[l],
                             preferred_element_type=jnp.float32)
                     + blin_ref[l:l + 1, :])
    out = jnp.maximum(out, 0.0)
    h = jnp.maximum(
        jnp.dot(out, wp1_ref[...], preferred_element_type=jnp.float32)
        + bp1_ref[...], 0.0)
    o_ref[...] = (
        jnp.dot(h, wp2_ref[...], preferred_element_type=jnp.float32)
        + bp2_ref[...])


def _head(pools, Wl, blin, Wp1, bp1, Wp2, bp2):
    return pl.pallas_call(
        _head_body,
        out_shape=jax.ShapeDtypeStruct((G, 1), jnp.float32),
    )(*pools, Wl, blin, Wp1, bp1, Wp2, bp2)


def _decode_sc(o1d, aid, bid):
    per_w = LQ // NW

    @functools.partial(
        pl.kernel,
        mesh=plsc.VectorSubcoreMesh(**_MESH),
        compiler_params=_SC_PARAMS,
        out_type=jax.ShapeDtypeStruct((LQ,), jnp.float32),
        scratch_types=[
            pltpu.VMEM((G,), jnp.float32),
            pltpu.VMEM((per_w,), jnp.int32),
            pltpu.VMEM((per_w,), jnp.int32),
            pltpu.VMEM((per_w,), jnp.float32),
        ],
    )
    def body(o_hbm, a_hbm, b_hbm, out_hbm, ov, av, bv, pv):
        wid = lax.axis_index("c") * NSUB + lax.axis_index("s")
        base = wid * per_w
        pltpu.sync_copy(o_hbm, ov)
        pltpu.sync_copy(a_hbm.at[pl.ds(base, per_w)], av)
        pltpu.sync_copy(b_hbm.at[pl.ds(base, per_w)], bv)
        for g in range(per_w // 16):
            sl = pl.ds(g * 16, 16)
            va = plsc.load_gather(ov, [av[sl]])
            vb = plsc.load_gather(ov, [bv[sl]])
            pv[sl] = va * vb
        pltpu.sync_copy(pv, out_hbm.at[pl.ds(base, per_w)])

    return body(o1d, aid, bid)


def _pool_sc(y, batch_pad, neginf):
    @functools.partial(
        pl.kernel,
        mesh=plsc.VectorSubcoreMesh(**_MESH),
        compiler_params=_SC_PARAMS,
        out_type=jax.ShapeDtypeStruct((G, D), jnp.float32),
        scratch_types=[
            pltpu.VMEM((NPAD,), jnp.int32),
            pltpu.VMEM((64, D), jnp.float32),
            pltpu.VMEM((GPW, D), jnp.float32),
        ],
    )
    def body(y_hbm, b_hbm, ninf_hbm, out_hbm, bvec, ychunk, acc):
        wid = lax.axis_index("c") * NSUB + lax.axis_index("s")
        g0 = wid * GPW
        pltpu.sync_copy(b_hbm, bvec)
        pltpu.sync_copy(ninf_hbm, acc)

        def cbody(t, carry):
            lo, hi = carry
            v = bvec[pl.ds(t * 16, 16)]
            lo = lo + jnp.sum((v < g0).astype(jnp.int32))
            hi = hi + jnp.sum((v < g0 + GPW).astype(jnp.int32))
            return (lo, hi)

        r_lo, r_hi = lax.fori_loop(0, NPAD // 16, cbody,
                                   (jnp.int32(0), jnp.int32(0)))

        lanes = [lax.iota(jnp.int32, 16) + (k * 16) for k in range(8)]

        def chunk_body(c, _):
            rbase = c * 64
            pltpu.sync_copy(y_hbm.at[pl.ds(rbase, 64)], ychunk)
            i_lo = jnp.maximum(r_lo - rbase, 0)
            i_hi = jnp.minimum(r_hi - rbase, 64)

            def row_body(i, _):
                r = rbase + i
                gv = plsc.load_gather(bvec, [jnp.full((16,), r, jnp.int32)])
                grow = gv - g0
                off = jnp.full((16,), i, jnp.int32)
                for k in range(8):
                    yv = plsc.load_gather(ychunk, [off, lanes[k]])
                    av = plsc.load_gather(acc, [grow, lanes[k]])
                    plsc.store_scatter(acc, [grow, lanes[k]],
                                       jnp.maximum(av, yv))
                return 0

            lax.fori_loop(i_lo, i_hi, row_body, 0)
            return 0

        lax.fori_loop(r_lo // 64, (r_hi + 63) // 64, chunk_body, 0)
        pltpu.sync_copy(acc, out_hbm.at[pl.ds(g0, GPW)])

    return body(y, batch_pad, neginf)


def _msg_sc(xf, e, idx3, offs, zrows):
    @functools.partial(
        pl.kernel,
        mesh=plsc.VectorSubcoreMesh(**_MESH),
        compiler_params=_SC_PARAMS,
        out_type=jax.ShapeDtypeStruct((NPAD, D), jnp.float32),
        scratch_types=[
            pltpu.VMEM((BROWS, D), jnp.float32),
            pltpu.VMEM((40,), jnp.int32),
            pltpu.VMEM((3, ECHUNK), jnp.int32),
            pltpu.VMEM((3, ECHUNK), jnp.int32),
            pltpu.VMEM((ECHUNK, D), jnp.float32),
            pltpu.VMEM((ECHUNK, D), jnp.float32),
            pltpu.VMEM((ECHUNK, D), jnp.float32),
            pltpu.VMEM((ECHUNK, D), jnp.float32),
            pltpu.SemaphoreType.DMA,
            pltpu.SemaphoreType.DMA,
            pltpu.SemaphoreType.DMA,
            pltpu.SemaphoreType.DMA,
            pltpu.SemaphoreType.DMA,
            pltpu.SemaphoreType.DMA,
        ],
    )
    def body(xf_hbm, e_hbm, idx3_hbm, offs_hbm, z_hbm, out_hbm,
             acc, offv, ib0, ib1, xr0, xr1, er0, er1,
             gi0, gi1, gx0, gx1, ge0, ge1):
        cid = lax.axis_index("c")
        sid = lax.axis_index("s")
        wid = cid * NSUB + sid
        row_lo = wid * BROWS
        pltpu.sync_copy(z_hbm, acc)
        pltpu.sync_copy(offs_hbm, offv)
        ovec = plsc.load_gather(offv, [jnp.full((16,), wid, jnp.int32)])
        off_lo = jnp.max(ovec)
        ovec1 = plsc.load_gather(offv, [jnp.full((16,), wid + 1, jnp.int32)])
        off_hi = jnp.max(ovec1)
        c0 = off_lo // ECHUNK
        nch = (off_hi + ECHUNK - 1) // ECHUNK - c0

        ib = (ib0, ib1)
        gi = (gi0, gi1)
        xr = (xr0, xr1)
        er = (er0, er1)
        gx = (gx0, gx1)
        ge = (ge0, ge1)
        lanes = [lax.iota(jnp.int32, 16) + (k * 16) for k in range(8)]
        two = jnp.full((16,), 2, jnp.int32)
        lo_v = jnp.full((16,), row_lo, jnp.int32)
        hi_v = jnp.full((16,), row_lo + BROWS, jnp.int32)

        def start_idx(t, j):
            pltpu.async_copy(idx3_hbm.at[c0 + t], ib[j], gi[j])

        def wait_idx(t, j):
            pltpu.make_async_copy(idx3_hbm.at[c0 + t], ib[j], gi[j]).wait()

        def start_loads(t, b):
            pltpu.async_copy(xf_hbm.at[ib[b].at[0]], xr[b], gx[b])
            pltpu.async_copy(
                e_hbm.at[pl.ds((c0 + t) * ECHUNK, ECHUNK)], er[b], ge[b])

        def wait_loads(t, b):
            pltpu.make_async_copy(xf_hbm.at[ib[b].at[0]], xr[b],
                                  gx[b]).wait()
            pltpu.make_async_copy(
                e_hbm.at[pl.ds((c0 + t) * ECHUNK, ECHUNK)], er[b],
                ge[b]).wait()

        def compute(b):
            xrb, erb, ibj = xr[b], er[b], ib[b]

            def row(r, _):
                rv = jnp.full((16,), r, jnp.int32)
                dstv = plsc.load_gather(ibj, [jnp.full((16,), 1, jnp.int32),
                                              rv])
                mask = (dstv >= lo_v) & (dstv < hi_v)
                drow = jnp.minimum(jnp.maximum(dstv - lo_v, 0), BROWS - 1)
                wbits = plsc.load_gather(ibj, [two, rv])
                wvec = plsc.bitcast(wbits, jnp.float32)
                for k in range(8):
                    sl = pl.ds(k * 16, 16)
                    mv = jnp.maximum(xrb[r, sl] + erb[r, sl], 0.0) * wvec
                    plsc.addupdate_scatter(acc, [drow, lanes[k]], mv,
                                           mask=mask)
                return 0

            lax.fori_loop(0, ECHUNK, row, 0)

        def phase(t, ibase):
            b = ibase
            nb = 1 - ibase
            pl.when(t + 1 < nch)(lambda: (wait_idx(t + 1, nb),
                                          start_loads(t + 1, nb))[0])
            wait_loads(t, b)
            compute(b)
            pl.when(t + 2 < nch)(lambda: start_idx(t + 2, b))

        def loop_body(t, _):
            lax.cond(t % 2 == 0, lambda: phase(t, 0), lambda: phase(t, 1))
            return 0

        @pl.when(nch > 0)
        def _():
            pltpu.async_copy(idx3_hbm.at[c0], ib0, gi0).wait()
            start_loads(0, 0)
            pl.when(nch > 1)(lambda: start_idx(1, 1))
            lax.fori_loop(0, nch, loop_body, 0)

        pltpu.sync_copy(acc, out_hbm.at[pl.ds(row_lo, BROWS)])

    return body(xf, e, idx3, offs, zrows)


def kernel(x, edge_index, edge_attr, edge_weight, batch, edge_index_labeled,
           edge_label, W_enc, b_enc, W_init, b_init, W_edge, b_edge, W1, b1,
           g1, be1, W2, b2, g2, be2, eps, W_lin, b_lin, Wp1, bp1, Wp2, bp2):
    f32 = jnp.float32
    xp = jnp.pad(x, ((0, NPAD - N), (0, 0)))
    batch_pad = jnp.pad(batch, (0, NPAD - N), constant_values=G)
    srcp = jnp.pad(edge_index[0], (0, EPAD - E))
    dstp = jnp.pad(edge_index[1], (0, EPAD - E))
    wp = jnp.pad(edge_weight, (0, EPAD - E))
    eap = jnp.pad(edge_attr, ((0, EPAD - E), (0, 0)))
    bucket = dstp // BROWS
    perm = jnp.argsort(bucket, stable=True)
    srcs = srcp[perm]
    dsts = dstp[perm]
    ws = wp[perm]
    eas = eap[perm]
    offs = jnp.searchsorted(bucket[perm],
                            jnp.arange(NW + 1, dtype=jnp.int32)
                            ).astype(jnp.int32)
    offs = jnp.pad(offs, (0, 40 - NW - 1))
    wbits = lax.bitcast_convert_type(ws, jnp.int32)
    idx3 = jnp.stack([srcs.reshape(EPAD // ECHUNK, ECHUNK),
                      dsts.reshape(EPAD // ECHUNK, ECHUNK),
                      wbits.reshape(EPAD // ECHUNK, ECHUNK)], axis=1)
    neginf = jnp.full((GPW, D), -3.0e38, f32)
    zrows = jnp.zeros((BROWS, D), f32)

    xf, y0 = _encoder(xp, W_enc, b_enc.reshape(1, D), W_init,
                      b_init.reshape(1, D))
    pools = [_pool_sc(y0, batch_pad, neginf)]
    for l in range(NL):
        el = _edge_mm(eas, W_edge[l], b_edge[l].reshape(1, D))
        aggr = _msg_sc(xf, el, idx3, offs, zrows)
        W1f = W1[l] * g1[l][None, :]
        b1f = (b1[l] * g1[l] + be1[l]).reshape(1, D)
        W2f = W2[l] * g2[l][None, :]
        b2f = (b2[l] * g2[l] + be2[l]).reshape(1, D)
        s = (1.0 + eps[l]).reshape(1)
        xf = _node_mlp(s, xf, aggr, W1f, b1f, W2f, b2f)
        pools.append(_pool_sc(xf, batch_pad, neginf))

    o = _head(pools, W_lin, b_lin, Wp1, bp1.reshape(1, D), Wp2,
              bp2.reshape(1, 1))
    pred = _decode_sc(o.reshape(G), edge_index_labeled[0],
                      edge_index_labeled[1])
    return pred, edge_label


# speedup vs baseline: 1.1378x; 1.1378x over previous
import functools

import jax
import jax.numpy as jnp
from jax import lax
from jax.experimental import pallas as pl
from jax.experimental.pallas import tpu as pltpu
from jax.experimental.pallas import tpu_sc as plsc

N = 10000
E = 160000
D = 128
DE = 16
G = 256
NL = 4
LQ = 1024

NPAD = 10240
EPAD = 163840
NCORE = 2
NSUB = 16
NW = NCORE * NSUB
ROWS_PER_SUB = NPAD // NSUB
EDGES_PER_W = EPAD // NW
ECHUNK = 128
NCHUNK = EDGES_PER_W // ECHUNK
GPW = G // NW
BROWS = NPAD // NW

_MESH = dict(core_axis_name="c", subcore_axis_name="s")
_SC_PARAMS = pltpu.CompilerParams(needs_layout_passes=False)


def _enc_body(x_ref, we_ref, be_ref, wi_ref, bi_ref, xf_ref, y0_ref):
    xf = jnp.maximum(
        jnp.dot(x_ref[...], we_ref[...], preferred_element_type=jnp.float32)
        + be_ref[...], 0.0)
    xf_ref[...] = xf
    y0_ref[...] = (
        jnp.dot(xf, wi_ref[...], preferred_element_type=jnp.float32)
        + bi_ref[...])


def _encoder(x, We, be, Wi, bi):
    blk = 512
    return pl.pallas_call(
        _enc_body,
        grid=(NPAD // blk,),
        in_specs=[
            pl.BlockSpec((blk, D), lambda i: (i, 0)),
            pl.BlockSpec((D, D), lambda i: (0, 0)),
            pl.BlockSpec((1, D), lambda i: (0, 0)),
            pl.BlockSpec((D, D), lambda i: (0, 0)),
            pl.BlockSpec((1, D), lambda i: (0, 0)),
        ],
        out_specs=[pl.BlockSpec((blk, D), lambda i: (i, 0)),
                   pl.BlockSpec((blk, D), lambda i: (i, 0))],
        out_shape=[jax.ShapeDtypeStruct((NPAD, D), jnp.float32),
                   jax.ShapeDtypeStruct((NPAD, D), jnp.float32)],
    )(x, We, be, Wi, bi)


def _edge_mm_body(ea_ref, w_ref, b_ref, e_ref):
    e_ref[...] = (
        jnp.dot(ea_ref[...], w_ref[...], preferred_element_type=jnp.float32)
        + b_ref[...])


def _edge_mm(ea, W, b):
    blk = 2048
    return pl.pallas_call(
        _edge_mm_body,
        grid=(EPAD // blk,),
        in_specs=[
            pl.BlockSpec((blk, DE), lambda i: (i, 0)),
            pl.BlockSpec((DE, D), lambda i: (0, 0)),
            pl.BlockSpec((1, D), lambda i: (0, 0)),
        ],
        out_specs=pl.BlockSpec((blk, D), lambda i: (i, 0)),
        out_shape=jax.ShapeDtypeStruct((EPAD, D), jnp.float32),
    )(ea, W, b)


def _node_body(s_ref, xf_ref, p_ref, w1_ref, b1_ref, w2_ref, b2_ref,
               o_ref):
    z = s_ref[0] * xf_ref[...] + p_ref[...]
    h = jnp.maximum(
        jnp.dot(z, w1_ref[...], preferred_element_type=jnp.float32)
        + b1_ref[...], 0.0)
    o_ref[...] = jnp.maximum(
        jnp.dot(h, w2_ref[...], preferred_element_type=jnp.float32)
        + b2_ref[...], 0.0)


def _node_mlp(s, xf, aggr, W1f, b1f, W2f, b2f):
    blk = 512
    nblk = NPAD // blk
    return pl.pallas_call(
        _node_body,
        grid=(nblk,),
        in_specs=[
            pl.BlockSpec(memory_space=pltpu.SMEM),
            pl.BlockSpec((blk, D), lambda i: (i, 0)),
            pl.BlockSpec((blk, D), lambda i: (i, 0)),
            pl.BlockSpec((D, D), lambda i: (0, 0)),
            pl.BlockSpec((1, D), lambda i: (0, 0)),
            pl.BlockSpec((D, D), lambda i: (0, 0)),
            pl.BlockSpec((1, D), lambda i: (0, 0)),
        ],
        out_specs=pl.BlockSpec((blk, D), lambda i: (i, 0)),
        out_shape=jax.ShapeDtypeStruct((NPAD, D), jnp.float32),
    )(s, xf, aggr, W1f, b1f, W2f, b2f)


def _head_body(p0_ref, p1_ref, p2_ref, p3_ref, p4_ref, wl_ref, blin_ref,
               wp1_ref, bp1_ref, wp2_ref, bp2_ref, o_ref):
    out = p0_ref[...]
    for l, pref in enumerate((p1_ref, p2_ref, p3_ref, p4_ref)):
        out = out + (jnp.dot(pref[...], wl_ref[l],
                             preferred_element_type=jnp.float32)
                     + blin_ref[l:l + 1, :])
    out = jnp.maximum(out, 0.0)
    h = jnp.maximum(
        jnp.dot(out, wp1_ref[...], preferred_element_type=jnp.float32)
        + bp1_ref[...], 0.0)
    o_ref[...] = (
        jnp.dot(h, wp2_ref[...], preferred_element_type=jnp.float32)
        + bp2_ref[...])


def _head(pools, Wl, blin, Wp1, bp1, Wp2, bp2):
    return pl.pallas_call(
        _head_body,
        out_shape=jax.ShapeDtypeStruct((G, 1), jnp.float32),
    )(*pools, Wl, blin, Wp1, bp1, Wp2, bp2)


def _decode_sc(o1d, aid, bid):
    per_w = LQ // NW

    @functools.partial(
        pl.kernel,
        mesh=plsc.VectorSubcoreMesh(**_MESH),
        compiler_params=_SC_PARAMS,
        out_type=jax.ShapeDtypeStruct((LQ,), jnp.float32),
        scratch_types=[
            pltpu.VMEM((G,), jnp.float32),
            pltpu.VMEM((per_w,), jnp.int32),
            pltpu.VMEM((per_w,), jnp.int32),
            pltpu.VMEM((per_w,), jnp.float32),
        ],
    )
    def body(o_hbm, a_hbm, b_hbm, out_hbm, ov, av, bv, pv):
        wid = lax.axis_index("c") * NSUB + lax.axis_index("s")
        base = wid * per_w
        pltpu.sync_copy(o_hbm, ov)
        pltpu.sync_copy(a_hbm.at[pl.ds(base, per_w)], av)
        pltpu.sync_copy(b_hbm.at[pl.ds(base, per_w)], bv)
        for g in range(per_w // 16):
            sl = pl.ds(g * 16, 16)
            va = plsc.load_gather(ov, [av[sl]])
            vb = plsc.load_gather(ov, [bv[sl]])
            pv[sl] = va * vb
        pltpu.sync_copy(pv, out_hbm.at[pl.ds(base, per_w)])

    return body(o1d, aid, bid)


def _pool_sc(y, batch_pad, neginf):
    @functools.partial(
        pl.kernel,
        mesh=plsc.VectorSubcoreMesh(**_MESH),
        compiler_params=_SC_PARAMS,
        out_type=jax.ShapeDtypeStruct((G, D), jnp.float32),
        scratch_types=[
            pltpu.VMEM((NPAD,), jnp.int32),
            pltpu.VMEM((64, D), jnp.float32),
            pltpu.VMEM((GPW, D), jnp.float32),
        ],
    )
    def body(y_hbm, b_hbm, ninf_hbm, out_hbm, bvec, ychunk, acc):
        wid = lax.axis_index("c") * NSUB + lax.axis_index("s")
        g0 = wid * GPW
        pltpu.sync_copy(b_hbm, bvec)
        pltpu.sync_copy(ninf_hbm, acc)

        def cbody(t, carry):
            lo, hi = carry
            v = bvec[pl.ds(t * 16, 16)]
            lo = lo + jnp.sum((v < g0).astype(jnp.int32))
            hi = hi + jnp.sum((v < g0 + GPW).astype(jnp.int32))
            return (lo, hi)

        r_lo, r_hi = lax.fori_loop(0, NPAD // 16, cbody,
                                   (jnp.int32(0), jnp.int32(0)))

        lanes = [lax.iota(jnp.int32, 16) + (k * 16) for k in range(8)]

        def chunk_body(c, _):
            rbase = c * 64
            pltpu.sync_copy(y_hbm.at[pl.ds(rbase, 64)], ychunk)
            i_lo = jnp.maximum(r_lo - rbase, 0)
            i_hi = jnp.minimum(r_hi - rbase, 64)

            def row_body(i, _):
                r = rbase + i
                gv = plsc.load_gather(bvec, [jnp.full((16,), r, jnp.int32)])
                grow = gv - g0
                off = jnp.full((16,), i, jnp.int32)
                for k in range(8):
                    yv = plsc.load_gather(ychunk, [off, lanes[k]])
                    av = plsc.load_gather(acc, [grow, lanes[k]])
                    plsc.store_scatter(acc, [grow, lanes[k]],
                                       jnp.maximum(av, yv))
                return 0

            lax.fori_loop(i_lo, i_hi, row_body, 0)
            return 0

        lax.fori_loop(r_lo // 64, (r_hi + 63) // 64, chunk_body, 0)
        pltpu.sync_copy(acc, out_hbm.at[pl.ds(g0, GPW)])

    return body(y, batch_pad, neginf)


def _msg_sc(xf, e, idx3, offs, zrows):
    @functools.partial(
        pl.kernel,
        mesh=plsc.VectorSubcoreMesh(**_MESH),
        compiler_params=_SC_PARAMS,
        out_type=jax.ShapeDtypeStruct((NPAD, D), jnp.float32),
        scratch_types=[
            pltpu.VMEM((BROWS, D), jnp.float32),
            pltpu.VMEM((40,), jnp.int32),
            pltpu.VMEM((3, ECHUNK), jnp.int32),
            pltpu.VMEM((3, ECHUNK), jnp.int32),
            pltpu.VMEM((ECHUNK, D), jnp.float32),
            pltpu.VMEM((ECHUNK, D), jnp.float32),
            pltpu.VMEM((ECHUNK, D), jnp.float32),
            pltpu.VMEM((ECHUNK, D), jnp.float32),
            pltpu.SemaphoreType.DMA,
            pltpu.SemaphoreType.DMA,
            pltpu.SemaphoreType.DMA,
            pltpu.SemaphoreType.DMA,
            pltpu.SemaphoreType.DMA,
            pltpu.SemaphoreType.DMA,
        ],
    )
    def body(xf_hbm, e_hbm, idx3_hbm, offs_hbm, z_hbm, out_hbm,
             acc, offv, ib0, ib1, xr0, xr1, er0, er1,
             gi0, gi1, gx0, gx1, ge0, ge1):
        cid = lax.axis_index("c")
        sid = lax.axis_index("s")
        wid = cid * NSUB + sid
        row_lo = wid * BROWS
        pltpu.sync_copy(z_hbm, acc)
        pltpu.sync_copy(offs_hbm, offv)
        ovec = plsc.load_gather(offv, [jnp.full((16,), wid, jnp.int32)])
        off_lo = jnp.max(ovec)
        ovec1 = plsc.load_gather(offv, [jnp.full((16,), wid + 1, jnp.int32)])
        off_hi = jnp.max(ovec1)
        c0 = off_lo // ECHUNK
        nch = (off_hi + ECHUNK - 1) // ECHUNK - c0

        ib = (ib0, ib1)
        gi = (gi0, gi1)
        xr = (xr0, xr1)
        er = (er0, er1)
        gx = (gx0, gx1)
        ge = (ge0, ge1)
        lanes = [lax.iota(jnp.int32, 16) + (k * 16) for k in range(8)]
        two = jnp.full((16,), 2, jnp.int32)
        lo_v = jnp.full((16,), row_lo, jnp.int32)
        hi_v = jnp.full((16,), row_lo + BROWS, jnp.int32)

        def start_idx(t, j):
            pltpu.async_copy(idx3_hbm.at[c0 + t], ib[j], gi[j])

        def wait_idx(t, j):
            pltpu.make_async_copy(idx3_hbm.at[c0 + t], ib[j], gi[j]).wait()

        def start_loads(t, b):
            pltpu.async_copy(xf_hbm.at[ib[b].at[0]], xr[b], gx[b])
            pltpu.async_copy(
                e_hbm.at[pl.ds((c0 + t) * ECHUNK, ECHUNK)], er[b], ge[b])

        def wait_loads(t, b):
            pltpu.make_async_copy(xf_hbm.at[ib[b].at[0]], xr[b],
                                  gx[b]).wait()
            pltpu.make_async_copy(
                e_hbm.at[pl.ds((c0 + t) * ECHUNK, ECHUNK)], er[b],
                ge[b]).wait()

        def compute(b):
            xrb, erb, ibj = xr[b], er[b], ib[b]

            def row(r, _):
                rv = jnp.full((16,), r, jnp.int32)
                dstv = plsc.load_gather(ibj, [jnp.full((16,), 1, jnp.int32),
                                              rv])
                mask = (dstv >= lo_v) & (dstv < hi_v)
                drow = jnp.minimum(jnp.maximum(dstv - lo_v, 0), BROWS - 1)
                wbits = plsc.load_gather(ibj, [two, rv])
                wvec = plsc.bitcast(wbits, jnp.float32)
                for k in range(8):
                    sl = pl.ds(k * 16, 16)
                    mv = jnp.maximum(xrb[r, sl] + erb[r, sl], 0.0) * wvec
                    plsc.addupdate_scatter(acc, [drow, lanes[k]], mv,
                                           mask=mask)
                return 0

            lax.fori_loop(0, ECHUNK, row, 0)

        def phase(t, ibase):
            b = ibase
            nb = 1 - ibase
            pl.when(t + 1 < nch)(lambda: (wait_idx(t + 1, nb),
                                          start_loads(t + 1, nb))[0])
            wait_loads(t, b)
            compute(b)
            pl.when(t + 2 < nch)(lambda: start_idx(t + 2, b))

        def loop_body(t, _):
            lax.cond(t % 2 == 0, lambda: phase(t, 0), lambda: phase(t, 1))
            return 0

        @pl.when(nch > 0)
        def _():
            pltpu.async_copy(idx3_hbm.at[c0], ib0, gi0).wait()
            start_loads(0, 0)
            pl.when(nch > 1)(lambda: start_idx(1, 1))
            lax.fori_loop(0, nch, loop_body, 0)

        pltpu.sync_copy(acc, out_hbm.at[pl.ds(row_lo, BROWS)])

    return body(xf, e, idx3, offs, zrows)


def kernel(x, edge_index, edge_attr, edge_weight, batch, edge_index_labeled,
           edge_label, W_enc, b_enc, W_init, b_init, W_edge, b_edge, W1, b1,
           g1, be1, W2, b2, g2, be2, eps, W_lin, b_lin, Wp1, bp1, Wp2, bp2):
    f32 = jnp.float32
    xp = jnp.pad(x, ((0, NPAD - N), (0, 0)))
    batch_pad = jnp.pad(batch, (0, NPAD - N), constant_values=G)
    srcp = jnp.pad(edge_index[0], (0, EPAD - E))
    pad_dst = (jnp.arange(EPAD - E, dtype=jnp.int32) % NW) * BROWS
    dstp = jnp.concatenate([edge_index[1], pad_dst])
    wp = jnp.pad(edge_weight, (0, EPAD - E))
    eap = jnp.pad(edge_attr, ((0, EPAD - E), (0, 0)))
    bucket = dstp // BROWS
    perm = jnp.argsort(bucket, stable=True)
    srcs = srcp[perm]
    dsts = dstp[perm]
    ws = wp[perm]
    eas = eap[perm]
    offs = jnp.searchsorted(bucket[perm],
                            jnp.arange(NW + 1, dtype=jnp.int32)
                            ).astype(jnp.int32)
    offs = jnp.pad(offs, (0, 40 - NW - 1))
    wbits = lax.bitcast_convert_type(ws, jnp.int32)
    idx3 = jnp.stack([srcs.reshape(EPAD // ECHUNK, ECHUNK),
                      dsts.reshape(EPAD // ECHUNK, ECHUNK),
                      wbits.reshape(EPAD // ECHUNK, ECHUNK)], axis=1)
    neginf = jnp.full((GPW, D), -3.0e38, f32)
    zrows = jnp.zeros((BROWS, D), f32)

    xf, y0 = _encoder(xp, W_enc, b_enc.reshape(1, D), W_init,
                      b_init.reshape(1, D))
    pools = [_pool_sc(y0, batch_pad, neginf)]
    for l in range(NL):
        el = _edge_mm(eas, W_edge[l], b_edge[l].reshape(1, D))
        aggr = _msg_sc(xf, el, idx3, offs, zrows)
        W1f = W1[l] * g1[l][None, :]
        b1f = (b1[l] * g1[l] + be1[l]).reshape(1, D)
        W2f = W2[l] * g2[l][None, :]
        b2f = (b2[l] * g2[l] + be2[l]).reshape(1, D)
        s = (1.0 + eps[l]).reshape(1)
        xf = _node_mlp(s, xf, aggr, W1f, b1f, W2f, b2f)
        pools.append(_pool_sc(xf, batch_pad, neginf))

    o = _head(pools, W_lin, b_lin, Wp1, bp1.reshape(1, D), Wp2,
              bp2.reshape(1, 1))
    pred = _decode_sc(o.reshape(G), edge_index_labeled[0],
                      edge_index_labeled[1])
    return pred, edge_label


# speedup vs baseline: 1.1421x; 1.0038x over previous
import functools

import jax
import jax.numpy as jnp
from jax import lax
from jax.experimental import pallas as pl
from jax.experimental.pallas import tpu as pltpu
from jax.experimental.pallas import tpu_sc as plsc

N = 10000
E = 160000
D = 128
DE = 16
G = 256
NL = 4
LQ = 1024

NPAD = 10240
EPAD = 163840
NCORE = 2
NSUB = 16
NW = NCORE * NSUB
ROWS_PER_SUB = NPAD // NSUB
EDGES_PER_W = EPAD // NW
ECHUNK = 128
NCHUNK = EDGES_PER_W // ECHUNK
GPW = G // NW
BROWS = NPAD // NW

_MESH = dict(core_axis_name="c", subcore_axis_name="s")
_SC_PARAMS = pltpu.CompilerParams(needs_layout_passes=False)


def _enc_body(x_ref, we_ref, be_ref, wi_ref, bi_ref, xf_ref, y0_ref):
    xf = jnp.maximum(
        jnp.dot(x_ref[...], we_ref[...], preferred_element_type=jnp.float32)
        + be_ref[...], 0.0)
    xf_ref[...] = xf
    y0_ref[...] = (
        jnp.dot(xf, wi_ref[...], preferred_element_type=jnp.float32)
        + bi_ref[...])


def _encoder(x, We, be, Wi, bi):
    blk = 512
    return pl.pallas_call(
        _enc_body,
        grid=(NPAD // blk,),
        in_specs=[
            pl.BlockSpec((blk, D), lambda i: (i, 0)),
            pl.BlockSpec((D, D), lambda i: (0, 0)),
            pl.BlockSpec((1, D), lambda i: (0, 0)),
            pl.BlockSpec((D, D), lambda i: (0, 0)),
            pl.BlockSpec((1, D), lambda i: (0, 0)),
        ],
        out_specs=[pl.BlockSpec((blk, D), lambda i: (i, 0)),
                   pl.BlockSpec((blk, D), lambda i: (i, 0))],
        out_shape=[jax.ShapeDtypeStruct((NPAD, D), jnp.float32),
                   jax.ShapeDtypeStruct((NPAD, D), jnp.float32)],
    )(x, We, be, Wi, bi)


def _edge_mm_body(ea_ref, w_ref, b_ref, e_ref):
    e_ref[...] = (
        jnp.dot(ea_ref[...], w_ref[...], preferred_element_type=jnp.float32)
        + b_ref[...])


def _edge_mm(ea, W, b):
    blk = 2048
    return pl.pallas_call(
        _edge_mm_body,
        grid=(EPAD // blk,),
        in_specs=[
            pl.BlockSpec((blk, DE), lambda i: (i, 0)),
            pl.BlockSpec((DE, D), lambda i: (0, 0)),
            pl.BlockSpec((1, D), lambda i: (0, 0)),
        ],
        out_specs=pl.BlockSpec((blk, D), lambda i: (i, 0)),
        out_shape=jax.ShapeDtypeStruct((EPAD, D), jnp.float32),
    )(ea, W, b)


def _node_body(s_ref, xf_ref, p_ref, w1_ref, b1_ref, w2_ref, b2_ref,
               o_ref):
    z = s_ref[0] * xf_ref[...] + p_ref[...]
    h = jnp.maximum(
        jnp.dot(z, w1_ref[...], preferred_element_type=jnp.float32)
        + b1_ref[...], 0.0)
    o_ref[...] = jnp.maximum(
        jnp.dot(h, w2_ref[...], preferred_element_type=jnp.float32)
        + b2_ref[...], 0.0)


def _node_mlp(s, xf, aggr, W1f, b1f, W2f, b2f):
    blk = 512
    nblk = NPAD // blk
    return pl.pallas_call(
        _node_body,
        grid=(nblk,),
        in_specs=[
            pl.BlockSpec(memory_space=pltpu.SMEM),
            pl.BlockSpec((blk, D), lambda i: (i, 0)),
            pl.BlockSpec((blk, D), lambda i: (i, 0)),
            pl.BlockSpec((D, D), lambda i: (0, 0)),
            pl.BlockSpec((1, D), lambda i: (0, 0)),
            pl.BlockSpec((D, D), lambda i: (0, 0)),
            pl.BlockSpec((1, D), lambda i: (0, 0)),
        ],
        out_specs=pl.BlockSpec((blk, D), lambda i: (i, 0)),
        out_shape=jax.ShapeDtypeStruct((NPAD, D), jnp.float32),
    )(s, xf, aggr, W1f, b1f, W2f, b2f)


def _head_body(p0_ref, p1_ref, p2_ref, p3_ref, p4_ref, wl_ref, blin_ref,
               wp1_ref, bp1_ref, wp2_ref, bp2_ref, o_ref):
    out = p0_ref[...]
    for l, pref in enumerate((p1_ref, p2_ref, p3_ref, p4_ref)):
        out = out + (jnp.dot(pref[...], wl_ref[l],
                             preferred_element_type=jnp.float32)
                     + blin_ref[l:l + 1, :])
    out = jnp.maximum(out, 0.0)
    h = jnp.maximum(
        jnp.dot(out, wp1_ref[...], preferred_element_type=jnp.float32)
        + bp1_ref[...], 0.0)
    o_ref[...] = (
        jnp.dot(h, wp2_ref[...], preferred_element_type=jnp.float32)
        + bp2_ref[...])


def _head(pools, Wl, blin, Wp1, bp1, Wp2, bp2):
    return pl.pallas_call(
        _head_body,
        out_shape=jax.ShapeDtypeStruct((G, 1), jnp.float32),
    )(*pools, Wl, blin, Wp1, bp1, Wp2, bp2)


def _decode_sc(o1d, aid, bid):
    per_w = LQ // NW

    @functools.partial(
        pl.kernel,
        mesh=plsc.VectorSubcoreMesh(**_MESH),
        compiler_params=_SC_PARAMS,
        out_type=jax.ShapeDtypeStruct((LQ,), jnp.float32),
        scratch_types=[
            pltpu.VMEM((G,), jnp.float32),
            pltpu.VMEM((per_w,), jnp.int32),
            pltpu.VMEM((per_w,), jnp.int32),
            pltpu.VMEM((per_w,), jnp.float32),
        ],
    )
    def body(o_hbm, a_hbm, b_hbm, out_hbm, ov, av, bv, pv):
        wid = lax.axis_index("c") * NSUB + lax.axis_index("s")
        base = wid * per_w
        pltpu.sync_copy(o_hbm, ov)
        pltpu.sync_copy(a_hbm.at[pl.ds(base, per_w)], av)
        pltpu.sync_copy(b_hbm.at[pl.ds(base, per_w)], bv)
        for g in range(per_w // 16):
            sl = pl.ds(g * 16, 16)
            va = plsc.load_gather(ov, [av[sl]])
            vb = plsc.load_gather(ov, [bv[sl]])
            pv[sl] = va * vb
        pltpu.sync_copy(pv, out_hbm.at[pl.ds(base, per_w)])

    return body(o1d, aid, bid)


def _pool_sc(y, batch_pad, neginf):
    @functools.partial(
        pl.kernel,
        mesh=plsc.VectorSubcoreMesh(**_MESH),
        compiler_params=_SC_PARAMS,
        out_type=jax.ShapeDtypeStruct((G, D), jnp.float32),
        scratch_types=[
            pltpu.VMEM((NPAD,), jnp.int32),
            pltpu.VMEM((64, D), jnp.float32),
            pltpu.VMEM((GPW, D), jnp.float32),
        ],
    )
    def body(y_hbm, b_hbm, ninf_hbm, out_hbm, bvec, ychunk, acc):
        wid = lax.axis_index("c") * NSUB + lax.axis_index("s")
        g0 = wid * GPW
        pltpu.sync_copy(b_hbm, bvec)
        pltpu.sync_copy(ninf_hbm, acc)

        def cbody(t, carry):
            lo, hi = carry
            v = bvec[pl.ds(t * 16, 16)]
            lo = lo + jnp.sum((v < g0).astype(jnp.int32))
            hi = hi + jnp.sum((v < g0 + GPW).astype(jnp.int32))
            return (lo, hi)

        r_lo, r_hi = lax.fori_loop(0, NPAD // 16, cbody,
                                   (jnp.int32(0), jnp.int32(0)))

        lanes = [lax.iota(jnp.int32, 16) + (k * 16) for k in range(8)]

        def chunk_body(c, _):
            rbase = c * 64
            pltpu.sync_copy(y_hbm.at[pl.ds(rbase, 64)], ychunk)
            i_lo = jnp.maximum(r_lo - rbase, 0)
            i_hi = jnp.minimum(r_hi - rbase, 64)

            def row_body(i, _):
                r = rbase + i
                gv = plsc.load_gather(bvec, [jnp.full((16,), r, jnp.int32)])
                grow = gv - g0
                off = jnp.full((16,), i, jnp.int32)
                for k in range(8):
                    yv = plsc.load_gather(ychunk, [off, lanes[k]])
                    av = plsc.load_gather(acc, [grow, lanes[k]])
                    plsc.store_scatter(acc, [grow, lanes[k]],
                                       jnp.maximum(av, yv))
                return 0

            lax.fori_loop(i_lo, i_hi, row_body, 0)
            return 0

        lax.fori_loop(r_lo // 64, (r_hi + 63) // 64, chunk_body, 0)
        pltpu.sync_copy(acc, out_hbm.at[pl.ds(g0, GPW)])

    return body(y, batch_pad, neginf)


def _msg_sc(xf, e, idx3, offs, zrows):
    @functools.partial(
        pl.kernel,
        mesh=plsc.VectorSubcoreMesh(**_MESH),
        compiler_params=_SC_PARAMS,
        out_type=jax.ShapeDtypeStruct((NPAD, D), jnp.float32),
        scratch_types=[
            pltpu.VMEM((BROWS, D), jnp.float32),
            pltpu.VMEM((40,), jnp.int32),
            pltpu.VMEM((3, ECHUNK), jnp.int32),
            pltpu.VMEM((3, ECHUNK), jnp.int32),
            pltpu.VMEM((ECHUNK, D), jnp.float32),
            pltpu.VMEM((ECHUNK, D), jnp.float32),
            pltpu.VMEM((ECHUNK, D), jnp.float32),
            pltpu.VMEM((ECHUNK, D), jnp.float32),
            pltpu.SemaphoreType.DMA,
            pltpu.SemaphoreType.DMA,
            pltpu.SemaphoreType.DMA,
            pltpu.SemaphoreType.DMA,
            pltpu.SemaphoreType.DMA,
            pltpu.SemaphoreType.DMA,
        ],
    )
    def body(xf_hbm, e_hbm, idx3_hbm, offs_hbm, z_hbm, out_hbm,
             acc, offv, ib0, ib1, xr0, xr1, er0, er1,
             gi0, gi1, gx0, gx1, ge0, ge1):
        cid = lax.axis_index("c")
        sid = lax.axis_index("s")
        wid = cid * NSUB + sid
        row_lo = wid * BROWS
        pltpu.sync_copy(z_hbm, acc)
        pltpu.sync_copy(offs_hbm, offv)
        ovec = plsc.load_gather(offv, [jnp.full((16,), wid, jnp.int32)])
        off_lo = jnp.max(ovec)
        ovec1 = plsc.load_gather(offv, [jnp.full((16,), wid + 1, jnp.int32)])
        off_hi = jnp.max(ovec1)
        c0 = off_lo // ECHUNK
        nch = (off_hi + ECHUNK - 1) // ECHUNK - c0

        ib = (ib0, ib1)
        gi = (gi0, gi1)
        xr = (xr0, xr1)
        er = (er0, er1)
        gx = (gx0, gx1)
        ge = (ge0, ge1)
        lanes = [lax.iota(jnp.int32, 16) + (k * 16) for k in range(8)]
        two = jnp.full((16,), 2, jnp.int32)
        lo_v = jnp.full((16,), row_lo, jnp.int32)
        hi_v = jnp.full((16,), row_lo + BROWS, jnp.int32)

        def start_idx(t, j):
            pltpu.async_copy(idx3_hbm.at[c0 + t], ib[j], gi[j])

        def wait_idx(t, j):
            pltpu.make_async_copy(idx3_hbm.at[c0 + t], ib[j], gi[j]).wait()

        def start_loads(t, b):
            pltpu.async_copy(xf_hbm.at[ib[b].at[0]], xr[b], gx[b])
            pltpu.async_copy(
                e_hbm.at[pl.ds((c0 + t) * ECHUNK, ECHUNK)], er[b], ge[b])

        def wait_loads(t, b):
            pltpu.make_async_copy(xf_hbm.at[ib[b].at[0]], xr[b],
                                  gx[b]).wait()
            pltpu.make_async_copy(
                e_hbm.at[pl.ds((c0 + t) * ECHUNK, ECHUNK)], er[b],
                ge[b]).wait()

        def compute(b):
            xrb, erb, ibj = xr[b], er[b], ib[b]

            one = jnp.full((16,), 1, jnp.int32)

            def rowpair(p, _):
                for u in range(2):
                    r = 2 * p + u
                    rv = jnp.full((16,), r, jnp.int32)
                    dstv = plsc.load_gather(ibj, [one, rv])
                    mask = (dstv >= lo_v) & (dstv < hi_v)
                    drow = jnp.minimum(jnp.maximum(dstv - lo_v, 0),
                                       BROWS - 1)
                    wbits = plsc.load_gather(ibj, [two, rv])
                    wvec = plsc.bitcast(wbits, jnp.float32)
                    for k in range(8):
                        sl = pl.ds(k * 16, 16)
                        mv = jnp.maximum(xrb[r, sl] + erb[r, sl],
                                         0.0) * wvec
                        plsc.addupdate_scatter(acc, [drow, lanes[k]], mv,
                                               mask=mask)
                return 0

            lax.fori_loop(0, ECHUNK // 2, rowpair, 0)

        def phase(t, ibase):
            b = ibase
            nb = 1 - ibase
            pl.when(t + 1 < nch)(lambda: (wait_idx(t + 1, nb),
                                          start_loads(t + 1, nb))[0])
            wait_loads(t, b)
            compute(b)
            pl.when(t + 2 < nch)(lambda: start_idx(t + 2, b))

        def loop_body(t, _):
            lax.cond(t % 2 == 0, lambda: phase(t, 0), lambda: phase(t, 1))
            return 0

        @pl.when(nch > 0)
        def _():
            pltpu.async_copy(idx3_hbm.at[c0], ib0, gi0).wait()
            start_loads(0, 0)
            pl.when(nch > 1)(lambda: start_idx(1, 1))
            lax.fori_loop(0, nch, loop_body, 0)

        pltpu.sync_copy(acc, out_hbm.at[pl.ds(row_lo, BROWS)])

    return body(xf, e, idx3, offs, zrows)


def kernel(x, edge_index, edge_attr, edge_weight, batch, edge_index_labeled,
           edge_label, W_enc, b_enc, W_init, b_init, W_edge, b_edge, W1, b1,
           g1, be1, W2, b2, g2, be2, eps, W_lin, b_lin, Wp1, bp1, Wp2, bp2):
    f32 = jnp.float32
    xp = jnp.pad(x, ((0, NPAD - N), (0, 0)))
    batch_pad = jnp.pad(batch, (0, NPAD - N), constant_values=G)
    srcp = jnp.pad(edge_index[0], (0, EPAD - E))
    pad_dst = (jnp.arange(EPAD - E, dtype=jnp.int32) % NW) * BROWS
    dstp = jnp.concatenate([edge_index[1], pad_dst])
    wp = jnp.pad(edge_weight, (0, EPAD - E))
    eap = jnp.pad(edge_attr, ((0, EPAD - E), (0, 0)))
    bucket = dstp // BROWS
    perm = jnp.argsort(bucket, stable=True)
    srcs = srcp[perm]
    dsts = dstp[perm]
    ws = wp[perm]
    eas = eap[perm]
    offs = jnp.searchsorted(bucket[perm],
                            jnp.arange(NW + 1, dtype=jnp.int32)
                            ).astype(jnp.int32)
    offs = jnp.pad(offs, (0, 40 - NW - 1))
    wbits = lax.bitcast_convert_type(ws, jnp.int32)
    idx3 = jnp.stack([srcs.reshape(EPAD // ECHUNK, ECHUNK),
                      dsts.reshape(EPAD // ECHUNK, ECHUNK),
                      wbits.reshape(EPAD // ECHUNK, ECHUNK)], axis=1)
    neginf = jnp.full((GPW, D), -3.0e38, f32)
    zrows = jnp.zeros((BROWS, D), f32)

    xf, y0 = _encoder(xp, W_enc, b_enc.reshape(1, D), W_init,
                      b_init.reshape(1, D))
    pools = [_pool_sc(y0, batch_pad, neginf)]
    for l in range(NL):
        el = _edge_mm(eas, W_edge[l], b_edge[l].reshape(1, D))
        aggr = _msg_sc(xf, el, idx3, offs, zrows)
        W1f = W1[l] * g1[l][None, :]
        b1f = (b1[l] * g1[l] + be1[l]).reshape(1, D)
        W2f = W2[l] * g2[l][None, :]
        b2f = (b2[l] * g2[l] + be2[l]).reshape(1, D)
        s = (1.0 + eps[l]).reshape(1)
        xf = _node_mlp(s, xf, aggr, W1f, b1f, W2f, b2f)
        pools.append(_pool_sc(xf, batch_pad, neginf))

    o = _head(pools, W_lin, b_lin, Wp1, bp1.reshape(1, D), Wp2,
              bp2.reshape(1, 1))
    pred = _decode_sc(o.reshape(G), edge_index_labeled[0],
                      edge_index_labeled[1])
    return pred, edge_label
